# baseline (speedup 1.0000x reference)
import functools

import jax
import jax.numpy as jnp
from jax import lax
from jax.experimental import pallas as pl
from jax.experimental.pallas import tpu as pltpu
from jax.experimental.pallas import tpu_sc as plsc

NW = 32
V = 1000000
D = 32
NB_FULL = 7808
SCR_ROWS = 250000
NSLOT = 4


def _mesh():
    return plsc.VectorSubcoreMesh(core_axis_name="c", subcore_axis_name="s")


def _make_retile():
    KS = 4
    SB = 7808 // (NW * KS)

    @functools.partial(
        pl.kernel,
        mesh=_mesh(),
        compiler_params=pltpu.CompilerParams(needs_layout_passes=False),
        out_type=jax.ShapeDtypeStruct((SCR_ROWS, 128), jnp.float32),
        scratch_types=[
            [pltpu.VMEM((32, 512), jnp.float32)] * 2,
            [pltpu.VMEM((128, 128), jnp.float32)] * 2,
            [pltpu.SemaphoreType.DMA] * 2,
            [pltpu.SemaphoreType.DMA] * 2,
        ],
    )
    def retile_kernel(tab_hbm, tail_hbm, scr_hbm, nats, trs, isems, xsems):
        wid = lax.axis_index("s") * 2 + lax.axis_index("c")
        rows_lo = lax.iota(jnp.int32, 16)
        rows_hi = rows_lo + 16

        def start_in(u, p):
            v0 = (wid + NW * u) * 512
            for tr in range(4):
                pltpu.async_copy(
                    tab_hbm.at[pl.ds(8 * tr, 8), pl.ds(v0, 512)],
                    nats[p].at[pl.ds(8 * tr, 8), :], isems[p])

        def wait_in(p):
            for tr in range(4):
                pltpu.make_async_copy(
                    tab_hbm.at[pl.ds(8 * tr, 8), pl.ds(0, 512)],
                    nats[p].at[pl.ds(8 * tr, 8), :], isems[p]).wait()

        def transpose_block(nat, tr, n_ids):
            @plsc.parallel_loop(0, n_ids, unroll=8)
            def _(u):
                colv = rows_lo * 0 + u
                a = plsc.load_gather(nat, [rows_lo, colv])
                b = plsc.load_gather(nat, [rows_hi, colv])
                row = u // 4
                col = (u % 4) * 32
                tr[row, pl.ds(col, 16)] = a
                tr[row, pl.ds(col + 16, 16)] = b

        def start_out(u, p):
            r0 = (wid + NW * u) * 128
            pltpu.async_copy(trs[p], scr_hbm.at[pl.ds(r0, 128), :], xsems[p])

        def wait_out(p):
            pltpu.make_async_copy(trs[p], scr_hbm.at[pl.ds(0, 128), :],
                                  xsems[p]).wait()

        start_in(0, 0)
        start_in(1, 1)

        def outer(j, _):
            for p in range(2):
                u = 2 * j + p

                @pl.when(u < SB)
                def _():
                    wait_in(p)

                    @pl.when(u >= 2)
                    def _():
                        wait_out(p)

                    transpose_block(nats[p], trs[p], 512)
                    start_out(u, p)

                    @pl.when(u + 2 < SB)
                    def _():
                        start_in(u + 2, p)

            return 0

        lax.fori_loop(0, (SB + 2) // 2, outer, 0)
        for p in range(2):
            wait_out(p)

        @pl.when(wid < 4)
        def _():
            i = 7808 + wid
            pltpu.sync_copy(tab_hbm.at[:, pl.ds(i * 128, 128)],
                            nats[0].at[:, pl.ds(0, 128)])
            transpose_block(nats[0], trs[0], 128)
            pltpu.sync_copy(trs[0].at[pl.ds(0, 32), :],
                            scr_hbm.at[pl.ds(i * 32, 32), :])

        @pl.when(wid == 4)
        def _():
            pltpu.sync_copy(tail_hbm, nats[0].at[:, pl.ds(0, 128)])
            transpose_block(nats[0], trs[0], 64)
            pltpu.sync_copy(trs[0].at[pl.ds(0, 16), :],
                            scr_hbm.at[pl.ds(249984, 16), :])

    return retile_kernel


def _make_gather(H, B):
    n_chunks = 50 * 4

    @functools.partial(
        pl.kernel,
        mesh=_mesh(),
        compiler_params=pltpu.CompilerParams(needs_layout_passes=False),
        out_type=jax.ShapeDtypeStruct((H, D, B), jnp.float32),
        scratch_types=[
            pltpu.VMEM((H, 512), jnp.int32),
            [pltpu.VMEM((128, 128), jnp.float32)] * NSLOT,
            [pltpu.VMEM((32, 128), jnp.float32)] * NSLOT,
            [pltpu.VMEM((1, 128), jnp.int32)] * NSLOT,
            [pltpu.VMEM((1, 128), jnp.int32)] * NSLOT,
            [pltpu.SemaphoreType.DMA] * NSLOT,
            [pltpu.SemaphoreType.DMA] * NSLOT,
        ],
    )
    def gather_kernel(ids_hbm, scr_hbm, out_hbm, ids_v, gbufs, trs,
                      idxs, offs, gsems, osems):
        wid = lax.axis_index("s") * 2 + lax.axis_index("c")
        iota = lax.iota(jnp.int32, 16)
        pltpu.sync_copy(ids_hbm.at[:, pl.ds(512 * wid, 512)], ids_v)

        def prep(t, p):
            h = lax.rem(t, 50)
            jj = t // 50
            for q in range(8):
                v = ids_v[h, pl.ds(128 * jj + 16 * q, 16)]
                idxs[p][0, pl.ds(16 * q, 16)] = lax.shift_right_logical(v, 2)
                offs[p][0, pl.ds(16 * q, 16)] = (v & 3) * 32

        def start_gather(p):
            pltpu.async_copy(scr_hbm.at[idxs[p].at[0]], gbufs[p], gsems[p])

        def wait_gather(p):
            pltpu.make_async_copy(scr_hbm.at[idxs[p].at[0]], gbufs[p],
                                  gsems[p]).wait()

        def process(t, p):
            gbuf, tr = gbufs[p], trs[p]
            rows_q = [iota + 16 * q for q in range(8)]
            offs_q = [offs[p][0, pl.ds(16 * q, 16)] for q in range(8)]

            @plsc.parallel_loop(0, 32, unroll=4)
            def _(f):
                for q in range(8):
                    val = plsc.load_gather(gbuf, [rows_q[q], offs_q[q] + f])
                    tr[f, pl.ds(16 * q, 16)] = val

            h = lax.rem(t, 50)
            jj = t // 50
            b0 = 512 * wid + 128 * jj
            pltpu.async_copy(tr, out_hbm.at[h, :, pl.ds(b0, 128)], osems[p])

        def wait_out(p):
            pltpu.make_async_copy(trs[p], out_hbm.at[0, :, pl.ds(0, 128)],
                                  osems[p]).wait()

        for p in range(NSLOT):
            prep(p, p)
            start_gather(p)

        n_rounds = n_chunks // NSLOT

        def outer(j, _):
            for p in range(NSLOT):
                t = NSLOT * j + p
                wait_gather(p)

                @pl.when(j >= 1)
                def _():
                    wait_out(p)

                process(t, p)

                @pl.when(j < n_rounds - 1)
                def _():
                    prep(t + NSLOT, p)
                    start_gather(p)

            return 0

        lax.fori_loop(0, n_rounds, outer, 0)
        for p in range(NSLOT):
            wait_out(p)

    return gather_kernel


def kernel(token_ids, embedding_table):
    Bt, H = token_ids.shape
    ids_t = token_ids.T.astype(jnp.int32)
    tab_t = embedding_table.T
    tail = jnp.zeros((D, 128), jnp.float32).at[:, :64].set(
        tab_t[:, V - 64:])
    scr = _make_retile()(tab_t, tail)
    out = _make_gather(H, Bt)(ids_t, scr)
    return out.transpose(2, 0, 1)


# speedup vs baseline: 1.0595x; 1.0595x over previous
import functools

import jax
import jax.numpy as jnp
from jax import lax
from jax.experimental import pallas as pl
from jax.experimental.pallas import tpu as pltpu
from jax.experimental.pallas import tpu_sc as plsc

NW = 32
V = 1000000
D = 32
NB_FULL = 7808
SCR_ROWS = 250000
NSLOT = 4


def _mesh():
    return plsc.VectorSubcoreMesh(core_axis_name="c", subcore_axis_name="s")


def _make_retile():
    KS = 4
    SB = 7808 // (NW * KS)

    @functools.partial(
        pl.kernel,
        mesh=_mesh(),
        compiler_params=pltpu.CompilerParams(needs_layout_passes=False),
        out_type=jax.ShapeDtypeStruct((SCR_ROWS, 128), jnp.float32),
        scratch_types=[
            [pltpu.VMEM((32, 512), jnp.float32)] * 2,
            [pltpu.VMEM((128, 128), jnp.float32)] * 2,
            [pltpu.SemaphoreType.DMA] * 2,
            [pltpu.SemaphoreType.DMA] * 2,
        ],
    )
    def retile_kernel(tab_hbm, tail_hbm, scr_hbm, nats, trs, isems, xsems):
        wid = lax.axis_index("s") * 2 + lax.axis_index("c")
        rows_lo = lax.iota(jnp.int32, 16)
        rows_hi = rows_lo + 16

        def start_in(u, p):
            v0 = (wid + NW * u) * 512
            for tr in range(4):
                pltpu.async_copy(
                    tab_hbm.at[pl.ds(8 * tr, 8), pl.ds(v0, 512)],
                    nats[p].at[pl.ds(8 * tr, 8), :], isems[p])

        def wait_in(p):
            for tr in range(4):
                pltpu.make_async_copy(
                    tab_hbm.at[pl.ds(8 * tr, 8), pl.ds(0, 512)],
                    nats[p].at[pl.ds(8 * tr, 8), :], isems[p]).wait()

        def transpose_block(nat, tr, n_ids):
            @plsc.parallel_loop(0, n_ids // 4, unroll=4)
            def _(r):
                base = rows_lo * 0 + 4 * r
                for m in range(4):
                    colv = base + m
                    a = plsc.load_gather(nat, [rows_lo, colv])
                    b = plsc.load_gather(nat, [rows_hi, colv])
                    tr[r, pl.ds(32 * m, 16)] = a
                    tr[r, pl.ds(32 * m + 16, 16)] = b

        def start_out(u, p):
            r0 = (wid + NW * u) * 128
            pltpu.async_copy(trs[p], scr_hbm.at[pl.ds(r0, 128), :], xsems[p])

        def wait_out(p):
            pltpu.make_async_copy(trs[p], scr_hbm.at[pl.ds(0, 128), :],
                                  xsems[p]).wait()

        start_in(0, 0)
        start_in(1, 1)

        def outer(j, _):
            for p in range(2):
                u = 2 * j + p

                @pl.when(u < SB)
                def _():
                    wait_in(p)

                    @pl.when(u >= 2)
                    def _():
                        wait_out(p)

                    transpose_block(nats[p], trs[p], 512)
                    start_out(u, p)

                    @pl.when(u + 2 < SB)
                    def _():
                        start_in(u + 2, p)

            return 0

        lax.fori_loop(0, (SB + 2) // 2, outer, 0)
        for p in range(2):
            wait_out(p)

        @pl.when(wid < 4)
        def _():
            i = 7808 + wid
            pltpu.sync_copy(tab_hbm.at[:, pl.ds(i * 128, 128)],
                            nats[0].at[:, pl.ds(0, 128)])
            transpose_block(nats[0], trs[0], 128)
            pltpu.sync_copy(trs[0].at[pl.ds(0, 32), :],
                            scr_hbm.at[pl.ds(i * 32, 32), :])

        @pl.when(wid == 4)
        def _():
            pltpu.sync_copy(tail_hbm, nats[0].at[:, pl.ds(0, 128)])
            transpose_block(nats[0], trs[0], 64)
            pltpu.sync_copy(trs[0].at[pl.ds(0, 16), :],
                            scr_hbm.at[pl.ds(249984, 16), :])

    return retile_kernel


def _make_gather(H, B):
    n_chunks = 50 * 4

    @functools.partial(
        pl.kernel,
        mesh=_mesh(),
        compiler_params=pltpu.CompilerParams(needs_layout_passes=False),
        out_type=jax.ShapeDtypeStruct((H, D, B), jnp.float32),
        scratch_types=[
            pltpu.VMEM((H, 512), jnp.int32),
            [pltpu.VMEM((128, 128), jnp.float32)] * NSLOT,
            [pltpu.VMEM((32, 128), jnp.float32)] * NSLOT,
            [pltpu.VMEM((1, 128), jnp.int32)] * NSLOT,
            [pltpu.VMEM((1, 128), jnp.int32)] * NSLOT,
            [pltpu.SemaphoreType.DMA] * NSLOT,
            [pltpu.SemaphoreType.DMA] * NSLOT,
        ],
    )
    def gather_kernel(ids_hbm, scr_hbm, out_hbm, ids_v, gbufs, trs,
                      idxs, offs, gsems, osems):
        wid = lax.axis_index("s") * 2 + lax.axis_index("c")
        iota = lax.iota(jnp.int32, 16)
        pltpu.sync_copy(ids_hbm.at[:, pl.ds(512 * wid, 512)], ids_v)

        def prep(t, p):
            h = lax.rem(t, 50)
            jj = t // 50
            for q in range(8):
                v = ids_v[h, pl.ds(128 * jj + 16 * q, 16)]
                idxs[p][0, pl.ds(16 * q, 16)] = lax.shift_right_logical(v, 2)
                offs[p][0, pl.ds(16 * q, 16)] = (v & 3) * 32

        def start_gather(p):
            pltpu.async_copy(scr_hbm.at[idxs[p].at[0]], gbufs[p], gsems[p])

        def wait_gather(p):
            pltpu.make_async_copy(scr_hbm.at[idxs[p].at[0]], gbufs[p],
                                  gsems[p]).wait()

        def process(t, p):
            gbuf, tr = gbufs[p], trs[p]
            rows_q = [iota + 16 * q for q in range(8)]
            offs_q = [offs[p][0, pl.ds(16 * q, 16)] for q in range(8)]

            @plsc.parallel_loop(0, 32, unroll=4)
            def _(f):
                for q in range(8):
                    val = plsc.load_gather(gbuf, [rows_q[q], offs_q[q] + f])
                    tr[f, pl.ds(16 * q, 16)] = val

            h = lax.rem(t, 50)
            jj = t // 50
            b0 = 512 * wid + 128 * jj
            pltpu.async_copy(tr, out_hbm.at[h, :, pl.ds(b0, 128)], osems[p])

        def wait_out(p):
            pltpu.make_async_copy(trs[p], out_hbm.at[0, :, pl.ds(0, 128)],
                                  osems[p]).wait()

        for p in range(NSLOT):
            prep(p, p)
            start_gather(p)

        n_rounds = n_chunks // NSLOT

        def outer(j, _):
            for p in range(NSLOT):
                t = NSLOT * j + p
                wait_gather(p)

                @pl.when(j >= 1)
                def _():
                    wait_out(p)

                process(t, p)

                @pl.when(j < n_rounds - 1)
                def _():
                    prep(t + NSLOT, p)
                    start_gather(p)

            return 0

        lax.fori_loop(0, n_rounds, outer, 0)
        for p in range(NSLOT):
            wait_out(p)

    return gather_kernel


def kernel(token_ids, embedding_table):
    Bt, H = token_ids.shape
    ids_t = token_ids.T.astype(jnp.int32)
    tab_t = embedding_table.T
    tail = jnp.zeros((D, 128), jnp.float32).at[:, :64].set(
        tab_t[:, V - 64:])
    scr = _make_retile()(tab_t, tail)
    out = _make_gather(H, Bt)(ids_t, scr)
    return out.transpose(2, 0, 1)


# speedup vs baseline: 1.1520x; 1.0873x over previous
import functools

import jax
import jax.numpy as jnp
from jax import lax
from jax.experimental import pallas as pl
from jax.experimental.pallas import tpu as pltpu
from jax.experimental.pallas import tpu_sc as plsc

NW = 32
V = 1000000
D = 32
NB_FULL = 7808
SCR_ROWS = 250000
NSLOT = 4


def _mesh():
    return plsc.VectorSubcoreMesh(core_axis_name="c", subcore_axis_name="s")


def _make_retile():
    KS = 4
    SB = 7808 // (NW * KS)

    @functools.partial(
        pl.kernel,
        mesh=_mesh(),
        compiler_params=pltpu.CompilerParams(needs_layout_passes=False),
        out_type=jax.ShapeDtypeStruct((SCR_ROWS, 128), jnp.float32),
        scratch_types=[
            [pltpu.VMEM((32, 512), jnp.float32)] * 2,
            [pltpu.VMEM((128, 128), jnp.float32)] * 2,
            [pltpu.SemaphoreType.DMA] * 2,
            [pltpu.SemaphoreType.DMA] * 2,
        ],
    )
    def retile_kernel(tab_hbm, tail_hbm, scr_hbm, nats, trs, isems, xsems):
        wid = lax.axis_index("s") * 2 + lax.axis_index("c")
        rows_lo = lax.iota(jnp.int32, 16)
        rows_hi = rows_lo + 16

        def start_in(u, p):
            v0 = (wid + NW * u) * 512
            for tr in range(4):
                pltpu.async_copy(
                    tab_hbm.at[pl.ds(8 * tr, 8), pl.ds(v0, 512)],
                    nats[p].at[pl.ds(8 * tr, 8), :], isems[p])

        def wait_in(p):
            for tr in range(4):
                pltpu.make_async_copy(
                    tab_hbm.at[pl.ds(8 * tr, 8), pl.ds(0, 512)],
                    nats[p].at[pl.ds(8 * tr, 8), :], isems[p]).wait()

        iota = rows_lo
        cols_b = [(iota & 3) * 32 + f for f in range(16, 32)]
        rows_b0 = lax.shift_right_logical(iota, 2)

        def transpose_block(nat, tr, n_ids):
            @plsc.parallel_loop(0, n_ids // 16, unroll=2)
            def _(g):
                u0 = g * 16
                u04 = g * 4
                base = iota * 0 + u0
                rows_sc = rows_b0 + u04
                for k in range(16):
                    a = plsc.load_gather(nat, [rows_lo, base + k])
                    tr[u04 + k // 4, pl.ds((k % 4) * 32, 16)] = a
                for f in range(16, 32):
                    x = nat[f, pl.ds(u0, 16)]
                    plsc.store_scatter(tr, [rows_sc, cols_b[f - 16]], x)

        def start_out(u, p):
            r0 = (wid + NW * u) * 128
            pltpu.async_copy(trs[p], scr_hbm.at[pl.ds(r0, 128), :], xsems[p])

        def wait_out(p):
            pltpu.make_async_copy(trs[p], scr_hbm.at[pl.ds(0, 128), :],
                                  xsems[p]).wait()

        start_in(0, 0)
        start_in(1, 1)

        def outer(j, _):
            for p in range(2):
                u = 2 * j + p

                @pl.when(u < SB)
                def _():
                    wait_in(p)

                    @pl.when(u >= 2)
                    def _():
                        wait_out(p)

                    transpose_block(nats[p], trs[p], 512)
                    start_out(u, p)

                    @pl.when(u + 2 < SB)
                    def _():
                        start_in(u + 2, p)

            return 0

        lax.fori_loop(0, (SB + 2) // 2, outer, 0)
        for p in range(2):
            wait_out(p)

        @pl.when(wid < 4)
        def _():
            i = 7808 + wid
            pltpu.sync_copy(tab_hbm.at[:, pl.ds(i * 128, 128)],
                            nats[0].at[:, pl.ds(0, 128)])
            transpose_block(nats[0], trs[0], 128)
            pltpu.sync_copy(trs[0].at[pl.ds(0, 32), :],
                            scr_hbm.at[pl.ds(i * 32, 32), :])

        @pl.when(wid == 4)
        def _():
            pltpu.sync_copy(tail_hbm, nats[0].at[:, pl.ds(0, 128)])
            transpose_block(nats[0], trs[0], 64)
            pltpu.sync_copy(trs[0].at[pl.ds(0, 16), :],
                            scr_hbm.at[pl.ds(249984, 16), :])

    return retile_kernel


def _make_gather(H, B):
    n_chunks = 50 * 4

    @functools.partial(
        pl.kernel,
        mesh=_mesh(),
        compiler_params=pltpu.CompilerParams(needs_layout_passes=False),
        out_type=jax.ShapeDtypeStruct((H, D, B), jnp.float32),
        scratch_types=[
            pltpu.VMEM((H, 512), jnp.int32),
            [pltpu.VMEM((128, 128), jnp.float32)] * NSLOT,
            [pltpu.VMEM((32, 128), jnp.float32)] * NSLOT,
            [pltpu.VMEM((1, 128), jnp.int32)] * NSLOT,
            [pltpu.VMEM((1, 128), jnp.int32)] * NSLOT,
            [pltpu.SemaphoreType.DMA] * NSLOT,
            [pltpu.SemaphoreType.DMA] * NSLOT,
        ],
    )
    def gather_kernel(ids_hbm, scr_hbm, out_hbm, ids_v, gbufs, trs,
                      idxs, offs, gsems, osems):
        wid = lax.axis_index("s") * 2 + lax.axis_index("c")
        iota = lax.iota(jnp.int32, 16)
        pltpu.sync_copy(ids_hbm.at[:, pl.ds(512 * wid, 512)], ids_v)

        def prep(t, p):
            h = lax.rem(t, 50)
            jj = t // 50
            for q in range(8):
                v = ids_v[h, pl.ds(128 * jj + 16 * q, 16)]
                idxs[p][0, pl.ds(16 * q, 16)] = lax.shift_right_logical(v, 2)
                offs[p][0, pl.ds(16 * q, 16)] = (v & 3) * 32

        def start_gather(p):
            pltpu.async_copy(scr_hbm.at[idxs[p].at[0]], gbufs[p], gsems[p])

        def wait_gather(p):
            pltpu.make_async_copy(scr_hbm.at[idxs[p].at[0]], gbufs[p],
                                  gsems[p]).wait()

        def process(t, p):
            gbuf, tr = gbufs[p], trs[p]
            rows_q = [iota + 16 * q for q in range(8)]
            offs_q = [offs[p][0, pl.ds(16 * q, 16)] for q in range(8)]

            @plsc.parallel_loop(0, 32, unroll=4)
            def _(f):
                for q in range(8):
                    val = plsc.load_gather(gbuf, [rows_q[q], offs_q[q] + f])
                    tr[f, pl.ds(16 * q, 16)] = val

            h = lax.rem(t, 50)
            jj = t // 50
            b0 = 512 * wid + 128 * jj
            pltpu.async_copy(tr, out_hbm.at[h, :, pl.ds(b0, 128)], osems[p])

        def wait_out(p):
            pltpu.make_async_copy(trs[p], out_hbm.at[0, :, pl.ds(0, 128)],
                                  osems[p]).wait()

        for p in range(NSLOT):
            prep(p, p)
            start_gather(p)

        n_rounds = n_chunks // NSLOT

        def outer(j, _):
            for p in range(NSLOT):
                t = NSLOT * j + p
                wait_gather(p)

                @pl.when(j >= 1)
                def _():
                    wait_out(p)

                process(t, p)

                @pl.when(j < n_rounds - 1)
                def _():
                    prep(t + NSLOT, p)
                    start_gather(p)

            return 0

        lax.fori_loop(0, n_rounds, outer, 0)
        for p in range(NSLOT):
            wait_out(p)

    return gather_kernel


def kernel(token_ids, embedding_table):
    Bt, H = token_ids.shape
    ids_t = token_ids.T.astype(jnp.int32)
    tab_t = embedding_table.T
    tail = jnp.zeros((D, 128), jnp.float32).at[:, :64].set(
        tab_t[:, V - 64:])
    scr = _make_retile()(tab_t, tail)
    out = _make_gather(H, Bt)(ids_t, scr)
    return out.transpose(2, 0, 1)
